# baseline (speedup 1.0000x reference)
import functools

import jax
import jax.numpy as jnp
from jax import lax
from jax.experimental import pallas as pl
from jax.experimental.pallas import tpu as pltpu
from jax.experimental.pallas import tpu_sc as plsc

N = 10000
E = 320000
D = 128
DH = D // 2
EPS = 1e-5

NC = 2
NS = 16
NW = NC * NS
CH = 40
EPW = E // NW
SCH = 128
NFULL = EPW // SCH
TAIL = EPW - NFULL * SCH
NCHT = E // CH
DEG_NCH = NCHT // NW
RPT = 624
RTAIL = N - NS * RPT
NPAD = 10240
DPT = NPAD // NS


def _mesh():
    return plsc.VectorSubcoreMesh(
        core_axis_name="c", subcore_axis_name="s", num_cores=NC, num_subcores=NS
    )


def _sc_degree_body(sd_hbm, degp_hbm, sd_v, ones_v, zero_v, deg_sh):
    cid = lax.axis_index("c")
    sid = lax.axis_index("s")
    wid = cid * NS + sid
    g0 = wid * DEG_NCH
    HB = DEG_NCH // 2
    pltpu.sync_copy(sd_hbm.at[pl.ds(g0, HB)], sd_v)
    for j in range(3):
        ones_v[pl.ds(16 * j, 16)] = jnp.ones((16,), jnp.float32)
    for j in range(DPT // 16):
        zero_v[pl.ds(16 * j, 16)] = jnp.zeros((16,), jnp.float32)
    pltpu.sync_copy(zero_v, deg_sh.at[pl.ds(sid * DPT, DPT)])
    plsc.subcore_barrier()

    def step(i, carry):
        pltpu.sync_copy(ones_v.at[pl.ds(0, CH)], deg_sh.at[sd_v.at[i, 1]], add=True)
        return carry

    lax.fori_loop(0, HB, step, 0)
    pltpu.sync_copy(sd_hbm.at[pl.ds(g0 + HB, HB)], sd_v)
    lax.fori_loop(0, HB, step, 0)
    plsc.subcore_barrier()
    pltpu.sync_copy(
        deg_sh.at[pl.ds(sid * DPT, DPT)],
        degp_hbm.at[pl.ds(cid * NPAD + sid * DPT, DPT)],
    )


@functools.cache
def _sc_degree():
    return pl.kernel(
        _sc_degree_body,
        out_type=jax.ShapeDtypeStruct((NC * NPAD,), jnp.float32),
        mesh=_mesh(),
        scratch_types=[
            pltpu.VMEM((DEG_NCH // 2, 2, CH), jnp.int32),
            pltpu.VMEM((48,), jnp.float32),
            pltpu.VMEM((DPT,), jnp.float32),
            pltpu.VMEM_SHARED((NPAD,), jnp.float32),
        ],
    )


def _sc_segsum_body(
    y_hbm, src_hbm, dst_hbm, part_hbm,
    idxs_v, idxd_v, idxt_s, idxt_d, rows_v, acc_sh,
    sem_i0, sem_i1, sem_i2, sem_i3, sem_i4, sem_i5,
    sem_r0, sem_r1, sem_s0, sem_s1,
):
    cid = lax.axis_index("c")
    sid = lax.axis_index("s")
    wid = cid * NS + sid
    e0 = wid * EPW
    sem_i = (sem_i0, sem_i1, sem_i2, sem_i3, sem_i4, sem_i5)
    sem_r = (sem_r0, sem_r1)
    sem_s = (sem_s0, sem_s1)

    pltpu.sync_copy(y_hbm.at[pl.ds(sid * RPT, RPT)], acc_sh.at[pl.ds(sid * RPT, RPT)])

    @pl.when(sid == 0)
    def _():
        pltpu.sync_copy(
            y_hbm.at[pl.ds(NS * RPT, RTAIL)], acc_sh.at[pl.ds(NS * RPT, RTAIL)]
        )

    plsc.subcore_barrier()

    def fa(i, q):
        base = pl.multiple_of(e0 + i * SCH, 8)
        pltpu.async_copy(src_hbm.at[pl.ds(base, SCH)], idxs_v.at[q], sem_i[q])
        pltpu.async_copy(dst_hbm.at[pl.ds(base, SCH)], idxd_v.at[q], sem_i[q])

    def iw(i, q):
        base = pl.multiple_of(e0 + i * SCH, 8)
        pltpu.make_async_copy(
            src_hbm.at[pl.ds(base, SCH)], idxs_v.at[q], sem_i[q]
        ).wait()
        pltpu.make_async_copy(
            dst_hbm.at[pl.ds(base, SCH)], idxd_v.at[q], sem_i[q]
        ).wait()

    def gs(q, r):
        pltpu.async_copy(y_hbm.at[idxs_v.at[q]], rows_v.at[r], sem_r[r])

    def gw(q, r):
        pltpu.make_async_copy(
            y_hbm.at[idxs_v.at[q]], rows_v.at[r], sem_r[r]
        ).wait()

    def ss(q, r):
        pltpu.async_copy(rows_v.at[r], acc_sh.at[idxd_v.at[q]], sem_s[r], add=True)

    def sw(q, r):
        pltpu.make_async_copy(
            rows_v.at[r], acc_sh.at[idxd_v.at[q]], sem_s[r]
        ).wait()

    fa(0, 0)
    fa(1, 1)
    fa(2, 2)
    fa(3, 3)
    iw(0, 0)
    gs(0, 0)

    def six(t, carry):
        a = 6 * t
        for k in range(6):
            c = a + k
            gw(k, k % 2)
            ss(k, k % 2)

            @pl.when(c + 1 < NFULL)
            def _(c=c, k=k):
                @pl.when(c >= 1)
                def _(c=c, k=k):
                    sw((k + 5) % 6, (k + 1) % 2)

                @pl.when(c + 4 < NFULL)
                def _(c=c, k=k):
                    fa(c + 4, (k + 4) % 6)

                iw(c + 1, (k + 1) % 6)
                gs((k + 1) % 6, (k + 1) % 2)
        return carry

    lax.fori_loop(0, NFULL // 6, six, 0)
    sw(4, 0)
    sw(5, 1)

    tbase = pl.multiple_of(e0 + NFULL * SCH, 8)
    pltpu.sync_copy(src_hbm.at[pl.ds(tbase, TAIL)], idxt_s)
    pltpu.sync_copy(dst_hbm.at[pl.ds(tbase, TAIL)], idxt_d)
    pltpu.sync_copy(y_hbm.at[idxt_s], rows_v.at[0, pl.ds(0, TAIL)])
    pltpu.sync_copy(rows_v.at[0, pl.ds(0, TAIL)], acc_sh.at[idxt_d], add=True)
    plsc.subcore_barrier()
    pltpu.sync_copy(
        acc_sh.at[pl.ds(sid * RPT, RPT)], part_hbm.at[cid, pl.ds(sid * RPT, RPT)]
    )

    @pl.when(sid == 0)
    def _():
        pltpu.sync_copy(
            acc_sh.at[pl.ds(NS * RPT, RTAIL)], part_hbm.at[cid, pl.ds(NS * RPT, RTAIL)]
        )


@functools.cache
def _sc_segsum():
    return pl.kernel(
        _sc_segsum_body,
        out_type=jax.ShapeDtypeStruct((NC, N, D), jnp.float32),
        mesh=_mesh(),
        scratch_types=[
            pltpu.VMEM((6, SCH), jnp.int32),
            pltpu.VMEM((6, SCH), jnp.int32),
            pltpu.VMEM((TAIL,), jnp.int32),
            pltpu.VMEM((TAIL,), jnp.int32),
            pltpu.VMEM((2, SCH, D), jnp.float32),
            pltpu.VMEM_SHARED((N, D), jnp.float32),
            pltpu.SemaphoreType.DMA,
            pltpu.SemaphoreType.DMA,
            pltpu.SemaphoreType.DMA,
            pltpu.SemaphoreType.DMA,
            pltpu.SemaphoreType.DMA,
            pltpu.SemaphoreType.DMA,
            pltpu.SemaphoreType.DMA,
            pltpu.SemaphoreType.DMA,
            pltpu.SemaphoreType.DMA,
            pltpu.SemaphoreType.DMA,
        ],
    )


def _tc_prescale_body(x_ref, w_ref, degp_ref, y_ref, dinv_ref):
    deg = degp_ref[0, :N] + degp_ref[1, :N] + 1.0
    dinv = lax.rsqrt(deg)
    dinv_ref[...] = dinv
    xw = jnp.dot(x_ref[...], w_ref[...], preferred_element_type=jnp.float32)
    y_ref[...] = dinv * xw


def _tc_mid_body(p_ref, y1_ref, dinv_ref, b1_ref, g_ref, be_ref, w2_ref, y_ref):
    dinv = dinv_ref[...]
    z = dinv * (p_ref[0] + p_ref[1] - y1_ref[...]) + b1_ref[...]
    mean = jnp.mean(z, axis=0, keepdims=True)
    zc = z - mean
    var = jnp.mean(zc * zc, axis=0, keepdims=True)
    h = jnp.maximum(g_ref[...] * zc * lax.rsqrt(var + EPS) + be_ref[...], 0.0)
    hw = jnp.dot(h, w2_ref[...], preferred_element_type=jnp.float32)
    y_ref[...] = dinv * hw


def _tc_final_body(p_ref, y2_ref, x_ref, dinv_ref, b2_ref, g_ref, be_ref, out_ref):
    z = dinv_ref[...] * (p_ref[0] + p_ref[1] - y2_ref[...]) + b2_ref[...]
    mean = jnp.mean(z, axis=0, keepdims=True)
    zc = z - mean
    var = jnp.mean(zc * zc, axis=0, keepdims=True)
    bn = g_ref[...] * zc * lax.rsqrt(var + EPS) + be_ref[...]
    out_ref[...] = jnp.maximum(bn + x_ref[...], 0.0)


def kernel(x, ei, batch, W1, b1, W2, b2, gamma2, beta2):
    del batch
    sd = ei.reshape(2, NCHT, CH).transpose(1, 0, 2)
    src = ei[0]
    dst = ei[1]
    b1r = b1.reshape(1, D)
    b2r = b2.reshape(1, D)
    gr = gamma2.reshape(1, D)
    ber = beta2.reshape(1, D)

    degp = _sc_degree()(sd)
    degp3 = degp.reshape(NC, NPAD, 1)

    y1, dinv = pl.pallas_call(
        _tc_prescale_body,
        out_shape=(
            jax.ShapeDtypeStruct((N, D), jnp.float32),
            jax.ShapeDtypeStruct((N, 1), jnp.float32),
        ),
    )(x, W1, degp3)

    p1 = _sc_segsum()(y1, src, dst)

    y2 = pl.pallas_call(
        _tc_mid_body,
        out_shape=jax.ShapeDtypeStruct((N, D), jnp.float32),
    )(p1, y1, dinv, b1r, gr, ber, W2)

    p2 = _sc_segsum()(y2, src, dst)

    out = pl.pallas_call(
        _tc_final_body,
        out_shape=jax.ShapeDtypeStruct((N, D), jnp.float32),
    )(p2, y2, x, dinv, b2r, gr, ber)
    return out


# speedup vs baseline: 1.0000x; 1.0000x over previous
import functools

import jax
import jax.numpy as jnp
from jax import lax
from jax.experimental import pallas as pl
from jax.experimental.pallas import tpu as pltpu
from jax.experimental.pallas import tpu_sc as plsc

N = 10000
E = 320000
D = 128
DH = D // 2
EPS = 1e-5

NC = 2
NS = 16
NW = NC * NS
CH = 40
EPW = E // NW
SCH = 128
NFULL = EPW // SCH
TAIL = EPW - NFULL * SCH
NCHT = E // CH
DEG_NCH = NCHT // NW
RPT = 624
RTAIL = N - NS * RPT
NPAD = 10240
DPT = NPAD // NS


def _mesh():
    return plsc.VectorSubcoreMesh(
        core_axis_name="c", subcore_axis_name="s", num_cores=NC, num_subcores=NS
    )


def _sc_degree_body(sd_hbm, degp_hbm, sd_v, ones_v, zero_v, deg_sh):
    cid = lax.axis_index("c")
    sid = lax.axis_index("s")
    wid = cid * NS + sid
    g0 = wid * DEG_NCH
    HB = DEG_NCH // 2
    pltpu.sync_copy(sd_hbm.at[pl.ds(g0, HB)], sd_v)
    for j in range(3):
        ones_v[pl.ds(16 * j, 16)] = jnp.ones((16,), jnp.float32)
    for j in range(DPT // 16):
        zero_v[pl.ds(16 * j, 16)] = jnp.zeros((16,), jnp.float32)
    pltpu.sync_copy(zero_v, deg_sh.at[pl.ds(sid * DPT, DPT)])
    plsc.subcore_barrier()

    def step(i, carry):
        pltpu.sync_copy(ones_v.at[pl.ds(0, CH)], deg_sh.at[sd_v.at[i, 1]], add=True)
        return carry

    lax.fori_loop(0, HB, step, 0)
    pltpu.sync_copy(sd_hbm.at[pl.ds(g0 + HB, HB)], sd_v)
    lax.fori_loop(0, HB, step, 0)
    plsc.subcore_barrier()
    pltpu.sync_copy(
        deg_sh.at[pl.ds(sid * DPT, DPT)],
        degp_hbm.at[pl.ds(cid * NPAD + sid * DPT, DPT)],
    )


@functools.cache
def _sc_degree():
    return pl.kernel(
        _sc_degree_body,
        out_type=jax.ShapeDtypeStruct((NC * NPAD,), jnp.float32),
        mesh=_mesh(),
        scratch_types=[
            pltpu.VMEM((DEG_NCH // 2, 2, CH), jnp.int32),
            pltpu.VMEM((48,), jnp.float32),
            pltpu.VMEM((DPT,), jnp.float32),
            pltpu.VMEM_SHARED((NPAD,), jnp.float32),
        ],
    )


def _sc_segsum_body(
    y_hbm, src_hbm, dst_hbm, part_hbm,
    idxs_v, idxd_v, idxt_s, idxt_d, rows_v, acc_sh,
    sem_i0, sem_i1, sem_i2, sem_i3, sem_i4, sem_i5,
    sem_r0, sem_r1, sem_s0, sem_s1,
):
    cid = lax.axis_index("c")
    sid = lax.axis_index("s")
    wid = cid * NS + sid
    e0 = wid * EPW
    sem_i = (sem_i0, sem_i1, sem_i2, sem_i3, sem_i4, sem_i5)
    sem_r = (sem_r0, sem_r1)
    sem_s = (sem_s0, sem_s1)

    pltpu.sync_copy(y_hbm.at[pl.ds(sid * RPT, RPT)], acc_sh.at[pl.ds(sid * RPT, RPT)])

    @pl.when(sid == 0)
    def _():
        pltpu.sync_copy(
            y_hbm.at[pl.ds(NS * RPT, RTAIL)], acc_sh.at[pl.ds(NS * RPT, RTAIL)]
        )

    plsc.subcore_barrier()

    def fa(i, q):
        base = pl.multiple_of(e0 + i * SCH, 8)
        pltpu.async_copy(src_hbm.at[pl.ds(base, SCH)], idxs_v.at[q], sem_i[q])
        pltpu.async_copy(dst_hbm.at[pl.ds(base, SCH)], idxd_v.at[q], sem_i[q])

    def iw(i, q):
        base = pl.multiple_of(e0 + i * SCH, 8)
        pltpu.make_async_copy(
            src_hbm.at[pl.ds(base, SCH)], idxs_v.at[q], sem_i[q]
        ).wait()
        pltpu.make_async_copy(
            dst_hbm.at[pl.ds(base, SCH)], idxd_v.at[q], sem_i[q]
        ).wait()

    def gs(q, r):
        pltpu.async_copy(y_hbm.at[idxs_v.at[q]], rows_v.at[r], sem_r[r])

    def gw(q, r):
        pltpu.make_async_copy(
            y_hbm.at[idxs_v.at[q]], rows_v.at[r], sem_r[r]
        ).wait()

    def ss(q, r):
        pltpu.async_copy(rows_v.at[r], acc_sh.at[idxd_v.at[q]], sem_s[r], add=True)

    def sw(q, r):
        pltpu.make_async_copy(
            rows_v.at[r], acc_sh.at[idxd_v.at[q]], sem_s[r]
        ).wait()

    fa(0, 0)
    fa(1, 1)
    fa(2, 2)
    fa(3, 3)
    iw(0, 0)
    gs(0, 0)

    def six(t, carry):
        a = 6 * t
        for k in range(6):
            c = a + k
            gw(k, k % 2)
            ss(k, k % 2)

            @pl.when(c + 1 < NFULL)
            def _(c=c, k=k):
                @pl.when(c >= 1)
                def _(c=c, k=k):
                    sw((k + 5) % 6, (k + 1) % 2)

                @pl.when(c + 4 < NFULL)
                def _(c=c, k=k):
                    fa(c + 4, (k + 4) % 6)

                iw(c + 1, (k + 1) % 6)
                gs((k + 1) % 6, (k + 1) % 2)
        return carry

    lax.fori_loop(0, NFULL // 6, six, 0)
    sw(4, 0)
    sw(5, 1)

    tbase = pl.multiple_of(e0 + NFULL * SCH, 8)
    pltpu.sync_copy(src_hbm.at[pl.ds(tbase, TAIL)], idxt_s)
    pltpu.sync_copy(dst_hbm.at[pl.ds(tbase, TAIL)], idxt_d)
    pltpu.sync_copy(y_hbm.at[idxt_s], rows_v.at[0, pl.ds(0, TAIL)])
    pltpu.sync_copy(rows_v.at[0, pl.ds(0, TAIL)], acc_sh.at[idxt_d], add=True)
    plsc.subcore_barrier()
    pltpu.sync_copy(
        acc_sh.at[pl.ds(sid * RPT, RPT)], part_hbm.at[cid, pl.ds(sid * RPT, RPT)]
    )

    @pl.when(sid == 0)
    def _():
        pltpu.sync_copy(
            acc_sh.at[pl.ds(NS * RPT, RTAIL)], part_hbm.at[cid, pl.ds(NS * RPT, RTAIL)]
        )


@functools.cache
def _sc_segsum():
    return pl.kernel(
        _sc_segsum_body,
        out_type=jax.ShapeDtypeStruct((NC, N, D), jnp.float32),
        mesh=_mesh(),
        scratch_types=[
            pltpu.VMEM((6, SCH), jnp.int32),
            pltpu.VMEM((6, SCH), jnp.int32),
            pltpu.VMEM((TAIL,), jnp.int32),
            pltpu.VMEM((TAIL,), jnp.int32),
            pltpu.VMEM((2, SCH, D), jnp.float32),
            pltpu.VMEM_SHARED((N, D), jnp.float32),
            pltpu.SemaphoreType.DMA,
            pltpu.SemaphoreType.DMA,
            pltpu.SemaphoreType.DMA,
            pltpu.SemaphoreType.DMA,
            pltpu.SemaphoreType.DMA,
            pltpu.SemaphoreType.DMA,
            pltpu.SemaphoreType.DMA,
            pltpu.SemaphoreType.DMA,
            pltpu.SemaphoreType.DMA,
            pltpu.SemaphoreType.DMA,
        ],
    )


def _tc_matmul_body(x_ref, w_ref, xw_ref):
    xw_ref[...] = jnp.dot(x_ref[...], w_ref[...], preferred_element_type=jnp.float32)


def _tc_prescale_body(xw_ref, degp_ref, y_ref, dinv_ref):
    deg = degp_ref[0, :N] + degp_ref[1, :N] + 1.0
    dinv = lax.rsqrt(deg)
    dinv_ref[...] = dinv
    y_ref[...] = dinv * xw_ref[...]


def _tc_mid_body(p_ref, y1_ref, dinv_ref, b1_ref, g_ref, be_ref, w2_ref, y_ref):
    dinv = dinv_ref[...]
    z = dinv * (p_ref[0] + p_ref[1] - y1_ref[...]) + b1_ref[...]
    mean = jnp.mean(z, axis=0, keepdims=True)
    zc = z - mean
    var = jnp.mean(zc * zc, axis=0, keepdims=True)
    h = jnp.maximum(g_ref[...] * zc * lax.rsqrt(var + EPS) + be_ref[...], 0.0)
    hw = jnp.dot(h, w2_ref[...], preferred_element_type=jnp.float32)
    y_ref[...] = dinv * hw


def _tc_final_body(p_ref, y2_ref, x_ref, dinv_ref, b2_ref, g_ref, be_ref, out_ref):
    z = dinv_ref[...] * (p_ref[0] + p_ref[1] - y2_ref[...]) + b2_ref[...]
    mean = jnp.mean(z, axis=0, keepdims=True)
    zc = z - mean
    var = jnp.mean(zc * zc, axis=0, keepdims=True)
    bn = g_ref[...] * zc * lax.rsqrt(var + EPS) + be_ref[...]
    out_ref[...] = jnp.maximum(bn + x_ref[...], 0.0)


def kernel(x, ei, batch, W1, b1, W2, b2, gamma2, beta2):
    del batch
    sd = ei.reshape(2, NCHT, CH).transpose(1, 0, 2)
    src = ei[0]
    dst = ei[1]
    b1r = b1.reshape(1, D)
    b2r = b2.reshape(1, D)
    gr = gamma2.reshape(1, D)
    ber = beta2.reshape(1, D)

    degp = _sc_degree()(sd)
    degp3 = degp.reshape(NC, NPAD, 1)

    xw1 = pl.pallas_call(
        _tc_matmul_body,
        out_shape=jax.ShapeDtypeStruct((N, D), jnp.float32),
    )(x, W1)

    y1, dinv = pl.pallas_call(
        _tc_prescale_body,
        out_shape=(
            jax.ShapeDtypeStruct((N, D), jnp.float32),
            jax.ShapeDtypeStruct((N, 1), jnp.float32),
        ),
    )(xw1, degp3)

    p1 = _sc_segsum()(y1, src, dst)

    y2 = pl.pallas_call(
        _tc_mid_body,
        out_shape=jax.ShapeDtypeStruct((N, D), jnp.float32),
    )(p1, y1, dinv, b1r, gr, ber, W2)

    p2 = _sc_segsum()(y2, src, dst)

    out = pl.pallas_call(
        _tc_final_body,
        out_shape=jax.ShapeDtypeStruct((N, D), jnp.float32),
    )(p2, y2, x, dinv, b2r, gr, ber)
    return out
